# baseline (speedup 1.0000x reference)
import functools

import jax
import jax.numpy as jnp
import numpy as np
from jax import lax
from jax.experimental import pallas as pl
from jax.experimental.pallas import tpu as pltpu
from jax.experimental.pallas import tpu_sc as plsc

B = 16384
F = 26
D = 16
VOCAB_PER_FIELD = 100000
N = B * F
NC, NS = 2, 16
NW = NC * NS
PER_W = N // NW
CHUNK = 1664
ROWS = CHUNK // F
CHUNKS = PER_W // CHUNK
D_IN = F * D
WIDE = 512
H1, H2 = 256, 128
BB = 1024


def _sc_gather(x_flat, emb_table, lin16, off_flat):
    mesh = plsc.VectorSubcoreMesh(core_axis_name="c", subcore_axis_name="s")

    @functools.partial(
        pl.kernel,
        out_type=jax.ShapeDtypeStruct((B, WIDE), jnp.float32),
        mesh=mesh,
        scratch_types=(
            pltpu.VMEM((CHUNK,), jnp.int32),
            pltpu.VMEM((CHUNK,), jnp.int32),
            pltpu.VMEM((CHUNK,), jnp.int32),
            pltpu.VMEM((CHUNK, D), jnp.float32),
            pltpu.VMEM((CHUNK, 16), jnp.float32),
            pltpu.VMEM((ROWS, WIDE), jnp.float32),
            pltpu.SemaphoreType.DMA,
            pltpu.SemaphoreType.DMA,
        ),
        compiler_params=pltpu.CompilerParams(use_tc_tiling_on_sc=False,
                                             needs_layout_passes=False),
    )
    def k(x_hbm, emb_hbm, lin_hbm, off_hbm, out_hbm,
          idxb, lidxb, offb, ebuf, lbuf, obuf, sem_e, sem_l):
        wid = lax.axis_index("s") * NC + lax.axis_index("c")
        base = pl.multiple_of(wid * PER_W, 8)
        row_base = pl.multiple_of(wid * (PER_W // F), 8)
        pltpu.sync_copy(off_hbm, offb)
        lane_iota = lax.iota(jnp.int32, 16)
        zeros16 = jnp.zeros((16,), jnp.float32)

        def zinit(r, c):
            for col in range(D_IN, WIDE, 16):
                obuf[r, pl.ds(col, 16)] = zeros16
            return c

        lax.fori_loop(0, ROWS, zinit, 0)

        def chunk(j, carry):
            s0 = pl.multiple_of(base + j * CHUNK, 8)
            pltpu.sync_copy(x_hbm.at[pl.ds(s0, CHUNK)], idxb)

            def add(i, c):
                s = pl.ds(pl.multiple_of(i * 16, 16), 16)
                xi = idxb[s] + offb[s]
                idxb[s] = xi
                lidxb[s] = lax.shift_right_logical(xi, 4)
                return c

            lax.fori_loop(0, CHUNK // 16, add, 0)
            ce = pltpu.async_copy(emb_hbm.at[idxb], ebuf, sem_e)
            cl = pltpu.async_copy(lin_hbm.at[lidxb], lbuf, sem_l)
            ce.wait()
            cl.wait()

            def sel(i, c):
                s = pl.ds(pl.multiple_of(i * 16, 16), 16)
                col = lax.bitwise_and(idxb[s], 15)
                row = lane_iota + i * 16
                vals = plsc.load_gather(lbuf, [row, col])
                p = lane_iota + i * 16
                orow = lax.div(p, jnp.int32(F))
                ocol = D_IN + lax.rem(p, jnp.int32(F))
                plsc.store_scatter(obuf, [orow, ocol], vals)
                return c

            lax.fori_loop(0, CHUNK // 16, sel, 0)

            def repack(r, c):
                for kf in range(F):
                    obuf[r, pl.ds(kf * 16, 16)] = ebuf[r * F + kf, :]
                return c

            lax.fori_loop(0, ROWS, repack, 0)
            r0 = pl.multiple_of(row_base + j * ROWS, 8)
            pltpu.sync_copy(obuf, out_hbm.at[pl.ds(r0, ROWS)])
            return carry

        lax.fori_loop(0, CHUNKS, chunk, 0)

    return k(x_flat, emb_table, lin16, off_flat)


def _tc_body(h_ref, w1_ref, b1_ref, w2_ref, b2_ref, w3_ref, b3_ref,
             s_ref, m_ref, out_ref):
    h = h_ref[...]
    se = jnp.dot(h, s_ref[...], preferred_element_type=jnp.float32)
    msel = jnp.dot(h * h, m_ref[...], preferred_element_type=jnp.float32)
    ysel = jnp.dot(h, m_ref[...], preferred_element_type=jnp.float32)
    sum_sq = msel[:, 0:1]
    ylin = ysel[:, 1:2]
    inter = 0.5 * (jnp.sum(se * se, axis=1, keepdims=True) - sum_sq)
    a = jnp.dot(h, w1_ref[...], preferred_element_type=jnp.float32) + b1_ref[...]
    a = jnp.maximum(a, 0.0)
    a = jnp.dot(a, w2_ref[...], preferred_element_type=jnp.float32) + b2_ref[...]
    a = jnp.maximum(a, 0.0)
    yd = jnp.dot(a, w3_ref[...], preferred_element_type=jnp.float32)
    out_ref[...] = yd + inter + ylin + b3_ref[...]


def _tc_mlp(h, W1p, b1, W2, b2, W3, b3c, Sp, Mp):
    grid = (B // BB,)
    return pl.pallas_call(
        _tc_body,
        grid=grid,
        in_specs=[
            pl.BlockSpec((BB, WIDE), lambda i: (i, 0)),
            pl.BlockSpec((WIDE, H1), lambda i: (0, 0)),
            pl.BlockSpec((1, H1), lambda i: (0, 0)),
            pl.BlockSpec((H1, H2), lambda i: (0, 0)),
            pl.BlockSpec((1, H2), lambda i: (0, 0)),
            pl.BlockSpec((H2, 1), lambda i: (0, 0)),
            pl.BlockSpec((1, 1), lambda i: (0, 0)),
            pl.BlockSpec((WIDE, D), lambda i: (0, 0)),
            pl.BlockSpec((WIDE, 2), lambda i: (0, 0)),
        ],
        out_specs=pl.BlockSpec((BB, 1), lambda i: (i, 0)),
        out_shape=jax.ShapeDtypeStruct((B, 1), jnp.float32),
    )(h, W1p, b1, W2, b2, W3, b3c, Sp, Mp)


def kernel(x, emb_table, lin_table, lin_bias, W1, b1, W2, b2, W3, b3):
    x_flat = x.reshape(N)
    pos = np.arange(CHUNK, dtype=np.int64)
    off_flat = jnp.asarray(((pos % F) * VOCAB_PER_FIELD).astype(np.int32))
    lin16 = lin_table.reshape(-1, 16)
    h = _sc_gather(x_flat, emb_table, lin16, off_flat)
    W1p = jnp.concatenate([W1, jnp.zeros((WIDE - D_IN, H1), jnp.float32)], axis=0)
    s_np = np.zeros((WIDE, D), np.float32)
    s_np[:D_IN] = np.tile(np.eye(D, dtype=np.float32), (F, 1))
    m_np = np.zeros((WIDE, 2), np.float32)
    m_np[:D_IN, 0] = 1.0
    m_np[D_IN:D_IN + F, 1] = 1.0
    y = _tc_mlp(h, W1p, b1.reshape(1, H1), W2, b2.reshape(1, H2), W3,
                (b3 + lin_bias).reshape(1, 1), jnp.asarray(s_np),
                jnp.asarray(m_np))
    return y.reshape(B)


# speedup vs baseline: 1.0040x; 1.0040x over previous
import functools

import jax
import jax.numpy as jnp
import numpy as np
from jax import lax
from jax.experimental import pallas as pl
from jax.experimental.pallas import tpu as pltpu
from jax.experimental.pallas import tpu_sc as plsc

B = 16384
F = 26
D = 16
VOCAB_PER_FIELD = 100000
N = B * F
NC, NS = 2, 16
NW = NC * NS
PER_W = N // NW
CHUNK = 1664
ROWS = CHUNK // F
CHUNKS = PER_W // CHUNK
D_IN = F * D
WIDE = 512
H1, H2 = 256, 128
BB = 1024


def _sc_gather(x_flat, emb_table, lin16, off_flat):
    mesh = plsc.VectorSubcoreMesh(core_axis_name="c", subcore_axis_name="s")

    @functools.partial(
        pl.kernel,
        out_type=jax.ShapeDtypeStruct((B, WIDE), jnp.float32),
        mesh=mesh,
        scratch_types=(
            pltpu.VMEM((CHUNK,), jnp.int32),
            pltpu.VMEM((CHUNK,), jnp.int32),
            pltpu.VMEM((CHUNK,), jnp.int32),
            pltpu.VMEM((CHUNK, D), jnp.float32),
            pltpu.VMEM((CHUNK, 16), jnp.float32),
            pltpu.VMEM((ROWS, WIDE), jnp.float32),
            pltpu.SemaphoreType.DMA,
            pltpu.SemaphoreType.DMA,
        ),
        compiler_params=pltpu.CompilerParams(use_tc_tiling_on_sc=False,
                                             needs_layout_passes=False),
    )
    def k(x_hbm, emb_hbm, lin_hbm, off_hbm, out_hbm,
          idxb, lidxb, offb, ebuf, lbuf, obuf, sem_e, sem_l):
        wid = lax.axis_index("s") * NC + lax.axis_index("c")
        base = pl.multiple_of(wid * PER_W, 8)
        row_base = pl.multiple_of(wid * (PER_W // F), 8)
        pltpu.sync_copy(off_hbm, offb)
        lane_iota = lax.iota(jnp.int32, 16)
        zeros16 = jnp.zeros((16,), jnp.float32)

        def zinit(r, c):
            for col in range(D_IN, WIDE, 16):
                obuf[r, pl.ds(col, 16)] = zeros16
            return c

        lax.fori_loop(0, ROWS, zinit, 0)

        def chunk(j, carry):
            s0 = pl.multiple_of(base + j * CHUNK, 8)
            pltpu.sync_copy(x_hbm.at[pl.ds(s0, CHUNK)], idxb)

            def add(i, c):
                s = pl.ds(pl.multiple_of(i * 16, 16), 16)
                xi = idxb[s] + offb[s]
                idxb[s] = xi
                lidxb[s] = lax.shift_right_logical(xi, 4)
                return c

            lax.fori_loop(0, CHUNK // 16, add, 0)
            ce = pltpu.async_copy(emb_hbm.at[idxb], ebuf, sem_e)
            cl = pltpu.async_copy(lin_hbm.at[lidxb], lbuf, sem_l)
            ce.wait()
            cl.wait()

            def sel(i, c):
                s = pl.ds(pl.multiple_of(i * 16, 16), 16)
                col = lax.bitwise_and(idxb[s], 15)
                row = lane_iota + i * 16
                vals = plsc.load_gather(lbuf, [row, col])
                p = lane_iota + i * 16
                orow = lax.div(p, jnp.int32(F))
                ocol = D_IN + lax.rem(p, jnp.int32(F))
                plsc.store_scatter(obuf, [orow, ocol], vals)
                return c

            lax.fori_loop(0, CHUNK // 16, sel, 0)

            def repack(r, c):
                for kf in range(F):
                    obuf[r, pl.ds(kf * 16, 16)] = ebuf[r * F + kf, :]
                return c

            lax.fori_loop(0, ROWS, repack, 0)
            r0 = pl.multiple_of(row_base + j * ROWS, 8)
            pltpu.sync_copy(obuf, out_hbm.at[pl.ds(r0, ROWS)])
            return carry

        lax.fori_loop(0, CHUNKS, chunk, 0)

    return k(x_flat, emb_table, lin16, off_flat)


def _tc_body(h_ref, w1_ref, b1_ref, w2_ref, b2_ref, w3_ref, b3_ref,
             s_ref, m_ref, out_ref):
    h = h_ref[...]
    se = jnp.dot(h, s_ref[...], preferred_element_type=jnp.float32)
    msel = jnp.dot(h * h, m_ref[...], preferred_element_type=jnp.float32)
    ysel = jnp.dot(h, m_ref[...], preferred_element_type=jnp.float32)
    sum_sq = msel[:, 0:1]
    ylin = ysel[:, 1:2]
    inter = 0.5 * (jnp.sum(se * se, axis=1, keepdims=True) - sum_sq)
    a = jnp.dot(h, w1_ref[...], preferred_element_type=jnp.float32) + b1_ref[...]
    a = jnp.maximum(a, 0.0)
    a = jnp.dot(a, w2_ref[...], preferred_element_type=jnp.float32) + b2_ref[...]
    a = jnp.maximum(a, 0.0)
    yd = jnp.dot(a, w3_ref[...], preferred_element_type=jnp.float32)
    out_ref[...] = yd + inter + ylin + b3_ref[...]


def _tc_mlp(h, W1p, b1, W2, b2, W3, b3c, Sp, Mp):
    grid = (B // BB,)
    return pl.pallas_call(
        _tc_body,
        grid=grid,
        in_specs=[
            pl.BlockSpec((BB, WIDE), lambda i: (i, 0)),
            pl.BlockSpec((WIDE, H1), lambda i: (0, 0)),
            pl.BlockSpec((1, H1), lambda i: (0, 0)),
            pl.BlockSpec((H1, H2), lambda i: (0, 0)),
            pl.BlockSpec((1, H2), lambda i: (0, 0)),
            pl.BlockSpec((H2, 1), lambda i: (0, 0)),
            pl.BlockSpec((1, 1), lambda i: (0, 0)),
            pl.BlockSpec((WIDE, D), lambda i: (0, 0)),
            pl.BlockSpec((WIDE, 2), lambda i: (0, 0)),
        ],
        out_specs=pl.BlockSpec((BB, 1), lambda i: (i, 0)),
        out_shape=jax.ShapeDtypeStruct((B, 1), jnp.float32),
    )(h, W1p, b1, W2, b2, W3, b3c, Sp, Mp)


def kernel(x, emb_table, lin_table, lin_bias, W1, b1, W2, b2, W3, b3):
    x_flat = x.reshape(N)
    pos = np.arange(CHUNK, dtype=np.int64)
    off_flat = jnp.asarray(((pos % F) * VOCAB_PER_FIELD).astype(np.int32))
    emb_wide = lax.optimization_barrier(emb_table.reshape(-1, 128))
    emb16 = emb_wide.reshape(-1, D)
    lin16 = lin_table.reshape(-1, 16)
    h = _sc_gather(x_flat, emb16, lin16, off_flat)
    W1p = jnp.concatenate([W1, jnp.zeros((WIDE - D_IN, H1), jnp.float32)], axis=0)
    s_np = np.zeros((WIDE, D), np.float32)
    s_np[:D_IN] = np.tile(np.eye(D, dtype=np.float32), (F, 1))
    m_np = np.zeros((WIDE, 2), np.float32)
    m_np[:D_IN, 0] = 1.0
    m_np[D_IN:D_IN + F, 1] = 1.0
    y = _tc_mlp(h, W1p, b1.reshape(1, H1), W2, b2.reshape(1, H2), W3,
                (b3 + lin_bias).reshape(1, 1), jnp.asarray(s_np),
                jnp.asarray(m_np))
    return y.reshape(B)
